# Initial kernel scaffold; baseline (speedup 1.0000x reference)
import jax
import jax.numpy as jnp
from jax.experimental import pallas as pl


def kernel(x, edge_index, W1, b1, W2, b2, Wf, bf, Wd1, bd1, Wd2, bd2, Wdf, bdf, Wl, bl):
    raise NotImplementedError("write your pallas kernel here")


# speedup vs baseline: 7.2621x; 7.2621x over previous
import functools

import jax
import jax.numpy as jnp
from jax import lax
from jax.experimental import pallas as pl
from jax.experimental.pallas import tpu as pltpu
from jax.experimental.pallas import tpu_sc as plsc

N_NODES = 10000
N_EDGES = 320000

NC = 2
NS = 16
K = 128
KE = 64

E_PAD = ((N_EDGES + NS * K - 1) // (NS * K)) * (NS * K)
EPT = E_PAD // NS
NCHUNK = EPT // K
EPW = E_PAD // (NC * NS)
NCHUNK_E = EPW // KE

PAD_ROWS = 16
NPS = 624
NREM = N_NODES - NS * NPS


def _sc_mesh():
    return plsc.VectorSubcoreMesh(core_axis_name="c", subcore_axis_name="s")


def _sc_params():
    return pltpu.CompilerParams(use_tc_tiling_on_sc=False, needs_layout_passes=False)


@functools.lru_cache(maxsize=None)
def _make_propagate(ch):
    @functools.partial(
        pl.kernel,
        out_type=jax.ShapeDtypeStruct((NC, N_NODES, ch), jnp.float32),
        mesh=_sc_mesh(),
        compiler_params=_sc_params(),
        scratch_types=[
            pltpu.VMEM_SHARED((N_NODES + PAD_ROWS, ch), jnp.float32),
            pltpu.VMEM((K,), jnp.int32),
            pltpu.VMEM((K,), jnp.int32),
            pltpu.VMEM((K, ch), jnp.float32),
            pltpu.SemaphoreType.DMA,
        ],
    )
    def prop(u_hbm, srcadj_hbm, dst_hbm, out_hbm, acc, idx_s, idx_d, rows, sem):
        c = lax.axis_index("c")
        s = lax.axis_index("s")
        pltpu.sync_copy(
            u_hbm.at[pl.ds(c * N_NODES + s * NPS, NPS)],
            acc.at[pl.ds(s * NPS, NPS)],
        )

        @pl.when(s == NS - 1)
        def _():
            pltpu.sync_copy(
                u_hbm.at[pl.ds(c * N_NODES + NS * NPS, NREM)],
                acc.at[pl.ds(NS * NPS, NREM)],
            )

        plsc.subcore_barrier()

        base = s * EPT

        def chunk(i, carry):
            off = base + i * K
            pltpu.sync_copy(srcadj_hbm.at[c, pl.ds(off, K)], idx_s)
            pltpu.sync_copy(dst_hbm.at[pl.ds(off, K)], idx_d)
            pltpu.async_copy(u_hbm.at[idx_s], rows, sem).wait()
            pltpu.sync_copy(rows, acc.at[idx_d], add=True)
            return carry

        lax.fori_loop(0, NCHUNK, chunk, 0)
        plsc.subcore_barrier()
        pltpu.sync_copy(
            acc.at[pl.ds(s * NPS, NPS)],
            out_hbm.at[c, pl.ds(s * NPS, NPS)],
        )

        @pl.when(s == NS - 1)
        def _():
            pltpu.sync_copy(
                acc.at[pl.ds(NS * NPS, NREM)],
                out_hbm.at[c, pl.ds(NS * NPS, NREM)],
            )

    return prop


def _make_edgeprob():
    @functools.partial(
        pl.kernel,
        out_type=jax.ShapeDtypeStruct((E_PAD,), jnp.float32),
        mesh=_sc_mesh(),
        compiler_params=_sc_params(),
        scratch_types=[
            pltpu.VMEM((N_NODES,), jnp.float32),
            pltpu.VMEM((N_NODES + PAD_ROWS,), jnp.float32),
            pltpu.VMEM((KE,), jnp.int32),
            pltpu.VMEM((KE,), jnp.int32),
            pltpu.VMEM((KE,), jnp.float32),
        ],
    )
    def eprob(pq_hbm, srcadj_hbm, dst_hbm, out_hbm, pv, qv, is_, id_, ob):
        c = lax.axis_index("c")
        s = lax.axis_index("s")
        w = s * NC + c
        pltpu.sync_copy(pq_hbm.at[0], pv)
        pltpu.sync_copy(pq_hbm.at[1], qv.at[pl.ds(0, N_NODES)])

        base = w * EPW

        def chunk(i, carry):
            off = base + i * KE
            pltpu.sync_copy(srcadj_hbm.at[0, pl.ds(off, KE)], is_)
            pltpu.sync_copy(dst_hbm.at[pl.ds(off, KE)], id_)

            def sub(j, carry2):
                sv = is_[pl.ds(j * 16, 16)]
                dv = id_[pl.ds(j * 16, 16)]
                a = plsc.load_gather(pv, [sv])
                b = plsc.load_gather(qv, [dv])
                t = a + b
                ob[pl.ds(j * 16, 16)] = 1.0 / (1.0 + jnp.exp(-t))
                return carry2

            lax.fori_loop(0, KE // 16, sub, 0)
            pltpu.sync_copy(ob, out_hbm.at[pl.ds(off, KE)])
            return carry

        lax.fori_loop(0, NCHUNK_E, chunk, 0)

    return eprob


BN = 1000


def _dinv(deg_ref):
    return lax.rsqrt(deg_ref[...][:, 0:1])


def _cat(y_ref):
    return jnp.concatenate([y_ref[0], y_ref[1]], axis=-1)


def _tc_call(body, in_arrays, in_specs, out_specs, out_shape):
    return pl.pallas_call(
        body,
        grid=(N_NODES // BN,),
        in_specs=in_specs,
        out_specs=out_specs,
        out_shape=out_shape,
    )(*in_arrays)


def _rowspec(c):
    return pl.BlockSpec((BN, c), lambda i: (i, 0))


def _fullspec(r, c):
    return pl.BlockSpec((r, c), lambda i: (0, 0))


def _splitspec(ch):
    return pl.BlockSpec((2, BN, ch), lambda i: (0, i, 0))


def _stage0(x, w1, deg):
    def body(x_ref, w_ref, deg_ref, out_ref):
        u = jnp.dot(x_ref[...], w_ref[...], preferred_element_type=jnp.float32)
        u = u * _dinv(deg_ref)
        out_ref[0] = u[:, :64]
        out_ref[1] = u[:, 64:]

    return _tc_call(
        body, (x, w1, deg),
        [_rowspec(128), _fullspec(128, 128), _rowspec(16)],
        _splitspec(64),
        jax.ShapeDtypeStruct((2, N_NODES, 64), jnp.float32),
    )


def _stage1(y1, deg, b1, w2):
    def body(y_ref, deg_ref, b_ref, w_ref, out_ref):
        di = _dinv(deg_ref)
        z1 = jnp.maximum(_cat(y_ref) * di + b_ref[...], 0.0)
        u2 = jnp.dot(z1, w_ref[...], preferred_element_type=jnp.float32) * di
        out_ref[0] = u2[:, :32]
        out_ref[1] = u2[:, 32:]

    return _tc_call(
        body, (y1, deg, b1, w2),
        [_splitspec(64), _rowspec(16), _fullspec(1, 128), _fullspec(128, 64)],
        _splitspec(32),
        jax.ShapeDtypeStruct((2, N_NODES, 32), jnp.float32),
    )


def _stage2(y2, deg, b2, wf, bf, wd1, wl, bl):
    def body(y_ref, deg_ref, b2_ref, wf_ref, bf_ref, wd1_ref, wl_ref, bl_ref,
             u3_ref, pq_ref):
        di = _dinv(deg_ref)
        z = _cat(y_ref) * di + b2_ref[...]
        z = jnp.dot(z, wf_ref[...], preferred_element_type=jnp.float32) + bf_ref[...]
        u3 = jnp.dot(z, wd1_ref[...], preferred_element_type=jnp.float32) * di
        u3_ref[0] = u3[:, :128]
        u3_ref[1] = u3[:, 128:]
        pq_ref[0] = jnp.dot(z, wl_ref[...][:64, :], preferred_element_type=jnp.float32)
        pq_ref[1] = jnp.dot(z, wl_ref[...][64:, :], preferred_element_type=jnp.float32) + bl_ref[...]

    return _tc_call(
        body, (y2, deg, b2, wf, bf, wd1, wl, bl),
        [_splitspec(32), _rowspec(16), _fullspec(1, 64), _fullspec(64, 64),
         _fullspec(1, 64), _fullspec(64, 256), _fullspec(128, 1), _fullspec(1, 1)],
        [_splitspec(128), pl.BlockSpec((2, BN, 1), lambda i: (0, i, 0))],
        [jax.ShapeDtypeStruct((2, N_NODES, 128), jnp.float32),
         jax.ShapeDtypeStruct((2, N_NODES, 1), jnp.float32)],
    )


def _stage3(y3, deg, bd1, wd2):
    def body(y_ref, deg_ref, b_ref, w_ref, out_ref):
        di = _dinv(deg_ref)
        h1 = jnp.maximum(_cat(y_ref) * di + b_ref[...], 0.0)
        u4 = jnp.dot(h1, w_ref[...], preferred_element_type=jnp.float32) * di
        out_ref[0] = u4[:, :64]
        out_ref[1] = u4[:, 64:]

    return _tc_call(
        body, (y3, deg, bd1, wd2),
        [_splitspec(128), _rowspec(16), _fullspec(1, 256), _fullspec(256, 128)],
        _splitspec(64),
        jax.ShapeDtypeStruct((2, N_NODES, 64), jnp.float32),
    )


def _stage4(y4, deg, bd2, wdf, bdf):
    def body(y_ref, deg_ref, b_ref, w_ref, bo_ref, out_ref):
        di = _dinv(deg_ref)
        h2 = _cat(y_ref) * di + b_ref[...]
        out_ref[...] = jnp.dot(h2, w_ref[...], preferred_element_type=jnp.float32) + bo_ref[...]

    return _tc_call(
        body, (y4, deg, bd2, wdf, bdf),
        [_splitspec(64), _rowspec(16), _fullspec(1, 128), _fullspec(128, 1024),
         _fullspec(1, 1024)],
        _rowspec(1024),
        jax.ShapeDtypeStruct((N_NODES, 1024), jnp.float32),
    )


def kernel(x, edge_index, W1, b1, W2, b2, Wf, bf, Wd1, bd1, Wd2, bd2, Wdf, bdf, Wl, bl):
    n = N_NODES
    src = edge_index[0].astype(jnp.int32)
    dst = edge_index[1].astype(jnp.int32)

    npad = E_PAD - N_EDGES
    pad_src = (jnp.arange(npad, dtype=jnp.int32)) % n
    pad_dst = n + (jnp.arange(npad, dtype=jnp.int32)) % PAD_ROWS
    srcp = jnp.concatenate([src, pad_src])
    dstp = jnp.concatenate([dst, pad_dst])
    srcadj = jnp.stack([srcp, srcp + n])

    b1r = b1.reshape(1, -1)
    b2r = b2.reshape(1, -1)
    bfr = bf.reshape(1, -1)
    bd1r = bd1.reshape(1, -1)
    bd2r = bd2.reshape(1, -1)
    bdfr = bdf.reshape(1, -1)
    blr = bl.reshape(1, 1)

    ones2 = jnp.ones((2 * n, 16), jnp.float32)
    deg = _make_propagate(16)(ones2, srcadj, dstp)[0]

    u1 = _stage0(x, W1, deg)
    y1 = _make_propagate(64)(u1.reshape(2 * n, 64), srcadj, dstp)
    u2 = _stage1(y1, deg, b1r, W2)
    y2 = _make_propagate(32)(u2.reshape(2 * n, 32), srcadj, dstp)
    u3, pq = _stage2(y2, deg, b2r, Wf, bfr, Wd1, Wl, blr)
    y3 = _make_propagate(128)(u3.reshape(2 * n, 128), srcadj, dstp)
    u4 = _stage3(y3, deg, bd1r, Wd2)
    y4 = _make_propagate(64)(u4.reshape(2 * n, 64), srcadj, dstp)
    x_hat = _stage4(y4, deg, bd2r, Wdf, bdfr)

    ep = _make_edgeprob()(pq.reshape(2, n), srcadj, dstp)
    edge_probs = ep[:N_EDGES].reshape(N_EDGES, 1)
    return (x_hat, edge_probs)


# speedup vs baseline: 18.2976x; 2.5196x over previous
import functools

import jax
import jax.numpy as jnp
from jax import lax
from jax.experimental import pallas as pl
from jax.experimental.pallas import tpu as pltpu
from jax.experimental.pallas import tpu_sc as plsc

N_NODES = 10000
N_EDGES = 320000

NC = 2
NS = 16
K = 128
NBUF = 2

NCHUNK = 160
E_PAD = NS * NCHUNK * K
EPT = E_PAD // NS
EPW = E_PAD // (NC * NS)

PAD_ROWS = 16
NPS = 624
NREM = N_NODES - NS * NPS


def _sc_mesh():
    return plsc.VectorSubcoreMesh(core_axis_name="c", subcore_axis_name="s")


def _sc_params():
    return pltpu.CompilerParams(use_tc_tiling_on_sc=False, needs_layout_passes=False)


@functools.lru_cache(maxsize=None)
def _make_propagate(ch, phases):
    nch_p = NCHUNK // phases

    @functools.partial(
        pl.kernel,
        out_type=jax.ShapeDtypeStruct((NC, N_NODES, ch), jnp.float32),
        mesh=_sc_mesh(),
        compiler_params=_sc_params(),
        scratch_types=[
            pltpu.VMEM_SHARED((N_NODES + PAD_ROWS, ch), jnp.float32),
            pltpu.VMEM((nch_p, K), jnp.int32),
            pltpu.VMEM((nch_p, K), jnp.int32),
        ]
        + [pltpu.VMEM((K, ch), jnp.float32) for _ in range(NBUF)]
        + [pltpu.SemaphoreType.DMA for _ in range(NBUF)],
    )
    def prop(u_hbm, srcadj_hbm, dst_hbm, out_hbm, acc, isall, idall, *rest):
        bufs = rest[:NBUF]
        sems = rest[NBUF:]
        c = lax.axis_index("c")
        s = lax.axis_index("s")
        pltpu.sync_copy(
            u_hbm.at[pl.ds(c * N_NODES + s * NPS, NPS)],
            acc.at[pl.ds(s * NPS, NPS)],
        )

        @pl.when(s == NS - 1)
        def _():
            pltpu.sync_copy(
                u_hbm.at[pl.ds(c * N_NODES + NS * NPS, NREM)],
                acc.at[pl.ds(NS * NPS, NREM)],
            )

        for h in range(phases):
            pltpu.sync_copy(srcadj_hbm.at[c, s, pl.ds(h * nch_p, nch_p)], isall)
            pltpu.sync_copy(dst_hbm.at[s, pl.ds(h * nch_p, nch_p)], idall)

            for b in range(NBUF):
                pltpu.async_copy(u_hbm.at[isall.at[b]], bufs[b], sems[b])

            if h == 0:
                plsc.subcore_barrier()

            def outer(g, carry):
                for b in range(NBUF):
                    i = g * NBUF + b
                    pltpu.make_async_copy(
                        u_hbm.at[isall.at[i]], bufs[b], sems[b]
                    ).wait()
                    pltpu.async_copy(
                        bufs[b], acc.at[idall.at[i]], sems[b], add=True
                    ).wait()

                    @pl.when(i + NBUF < nch_p)
                    def _():
                        pltpu.async_copy(
                            u_hbm.at[isall.at[i + NBUF]], bufs[b], sems[b]
                        )

                return carry

            lax.fori_loop(0, nch_p // NBUF, outer, 0)

        plsc.subcore_barrier()
        pltpu.sync_copy(
            acc.at[pl.ds(s * NPS, NPS)],
            out_hbm.at[c, pl.ds(s * NPS, NPS)],
        )

        @pl.when(s == NS - 1)
        def _():
            pltpu.sync_copy(
                acc.at[pl.ds(NS * NPS, NREM)],
                out_hbm.at[c, pl.ds(NS * NPS, NREM)],
            )

    return prop


def _make_degree():
    halfc = NCHUNK // NC

    @functools.partial(
        pl.kernel,
        out_type=jax.ShapeDtypeStruct((NC, N_NODES, 16), jnp.float32),
        mesh=_sc_mesh(),
        compiler_params=_sc_params(),
        scratch_types=[
            pltpu.VMEM_SHARED((N_NODES + PAD_ROWS, 16), jnp.float32),
            pltpu.VMEM((halfc, K), jnp.int32),
            pltpu.VMEM((K, 16), jnp.float32),
        ]
        + [pltpu.SemaphoreType.DMA for _ in range(NBUF)],
    )
    def degk(onesz_hbm, dst_hbm, out_hbm, acc, idall, ones, *sems):
        c = lax.axis_index("c")
        s = lax.axis_index("s")
        pltpu.sync_copy(
            onesz_hbm.at[pl.ds(c * N_NODES + s * NPS, NPS)],
            acc.at[pl.ds(s * NPS, NPS)],
        )

        @pl.when(s == NS - 1)
        def _():
            pltpu.sync_copy(
                onesz_hbm.at[pl.ds(c * N_NODES + NS * NPS, NREM)],
                acc.at[pl.ds(NS * NPS, NREM)],
            )

        pltpu.sync_copy(onesz_hbm.at[pl.ds(0, K)], ones)
        pltpu.sync_copy(dst_hbm.at[s, pl.ds(c * halfc, halfc)], idall)
        plsc.subcore_barrier()

        def outer(g, carry):
            for b in range(NBUF):
                i = g * NBUF + b

                @pl.when(g > 0)
                def _():
                    pltpu.make_async_copy(ones, acc.at[idall.at[i]], sems[b]).wait()

                pltpu.async_copy(ones, acc.at[idall.at[i]], sems[b], add=True)
            return carry

        lax.fori_loop(0, halfc // NBUF, outer, 0)
        for b in range(NBUF):
            pltpu.make_async_copy(ones, acc.at[idall.at[b]], sems[b]).wait()

        plsc.subcore_barrier()
        pltpu.sync_copy(
            acc.at[pl.ds(s * NPS, NPS)],
            out_hbm.at[c, pl.ds(s * NPS, NPS)],
        )

        @pl.when(s == NS - 1)
        def _():
            pltpu.sync_copy(
                acc.at[pl.ds(NS * NPS, NREM)],
                out_hbm.at[c, pl.ds(NS * NPS, NREM)],
            )

    return degk


def _make_edgeprob():
    @functools.partial(
        pl.kernel,
        out_type=jax.ShapeDtypeStruct((E_PAD,), jnp.float32),
        mesh=_sc_mesh(),
        compiler_params=_sc_params(),
        scratch_types=[
            pltpu.VMEM((N_NODES,), jnp.float32),
            pltpu.VMEM((N_NODES + PAD_ROWS,), jnp.float32),
            pltpu.VMEM((EPW,), jnp.int32),
            pltpu.VMEM((EPW,), jnp.int32),
            pltpu.VMEM((EPW,), jnp.float32),
        ],
    )
    def eprob(pq_hbm, src_hbm, dst_hbm, out_hbm, pv, qv, is_, id_, ob):
        c = lax.axis_index("c")
        s = lax.axis_index("s")
        w = s * NC + c
        pltpu.sync_copy(pq_hbm.at[0], pv)
        pltpu.sync_copy(pq_hbm.at[1], qv.at[pl.ds(0, N_NODES)])
        pltpu.sync_copy(src_hbm.at[w], is_)
        pltpu.sync_copy(dst_hbm.at[w], id_)

        def sub(j, carry):
            sv = is_[pl.ds(j * 16, 16)]
            dv = id_[pl.ds(j * 16, 16)]
            a = plsc.load_gather(pv, [sv])
            b = plsc.load_gather(qv, [dv])
            t = a + b
            ob[pl.ds(j * 16, 16)] = 1.0 / (1.0 + jnp.exp(-t))
            return carry

        lax.fori_loop(0, EPW // 16, sub, 0)
        pltpu.sync_copy(ob, out_hbm.at[pl.ds(w * EPW, EPW)])

    return eprob


BN = 1000


def _dinv(deg_ref):
    d = deg_ref[0][:, 0:1] + deg_ref[1][:, 0:1]
    return lax.rsqrt(d)


def _cat(y_ref):
    return jnp.concatenate([y_ref[0], y_ref[1]], axis=-1)


def _tc_call(body, in_arrays, in_specs, out_specs, out_shape):
    return pl.pallas_call(
        body,
        grid=(N_NODES // BN,),
        in_specs=in_specs,
        out_specs=out_specs,
        out_shape=out_shape,
    )(*in_arrays)


def _rowspec(c):
    return pl.BlockSpec((BN, c), lambda i: (i, 0))


def _fullspec(r, c):
    return pl.BlockSpec((r, c), lambda i: (0, 0))


def _splitspec(ch):
    return pl.BlockSpec((2, BN, ch), lambda i: (0, i, 0))


def _degspec():
    return pl.BlockSpec((2, BN, 16), lambda i: (0, i, 0))


def _stage0(x, w1, deg):
    def body(x_ref, w_ref, deg_ref, out_ref):
        u = jnp.dot(x_ref[...], w_ref[...], preferred_element_type=jnp.float32)
        u = u * _dinv(deg_ref)
        out_ref[0] = u[:, :64]
        out_ref[1] = u[:, 64:]

    return _tc_call(
        body, (x, w1, deg),
        [_rowspec(128), _fullspec(128, 128), _degspec()],
        _splitspec(64),
        jax.ShapeDtypeStruct((2, N_NODES, 64), jnp.float32),
    )


def _stage1(y1, deg, b1, w2):
    def body(y_ref, deg_ref, b_ref, w_ref, out_ref):
        di = _dinv(deg_ref)
        z1 = jnp.maximum(_cat(y_ref) * di + b_ref[...], 0.0)
        u2 = jnp.dot(z1, w_ref[...], preferred_element_type=jnp.float32) * di
        out_ref[0] = u2[:, :32]
        out_ref[1] = u2[:, 32:]

    return _tc_call(
        body, (y1, deg, b1, w2),
        [_splitspec(64), _degspec(), _fullspec(1, 128), _fullspec(128, 64)],
        _splitspec(32),
        jax.ShapeDtypeStruct((2, N_NODES, 32), jnp.float32),
    )


def _stage2(y2, deg, b2, wf, bf, wd1, wl, bl):
    def body(y_ref, deg_ref, b2_ref, wf_ref, bf_ref, wd1_ref, wl_ref, bl_ref,
             u3_ref, pq_ref):
        di = _dinv(deg_ref)
        z = _cat(y_ref) * di + b2_ref[...]
        z = jnp.dot(z, wf_ref[...], preferred_element_type=jnp.float32) + bf_ref[...]
        u3 = jnp.dot(z, wd1_ref[...], preferred_element_type=jnp.float32) * di
        u3_ref[0, 0] = u3[:, 0:64]
        u3_ref[1, 0] = u3[:, 64:128]
        u3_ref[0, 1] = u3[:, 128:192]
        u3_ref[1, 1] = u3[:, 192:256]
        pq_ref[0] = jnp.dot(z, wl_ref[...][:64, :], preferred_element_type=jnp.float32)
        pq_ref[1] = jnp.dot(z, wl_ref[...][64:, :], preferred_element_type=jnp.float32) + bl_ref[...]

    return _tc_call(
        body, (y2, deg, b2, wf, bf, wd1, wl, bl),
        [_splitspec(32), _degspec(), _fullspec(1, 64), _fullspec(64, 64),
         _fullspec(1, 64), _fullspec(64, 256), _fullspec(128, 1), _fullspec(1, 1)],
        [pl.BlockSpec((2, 2, BN, 64), lambda i: (0, 0, i, 0)),
         pl.BlockSpec((2, BN, 1), lambda i: (0, i, 0))],
        [jax.ShapeDtypeStruct((2, 2, N_NODES, 64), jnp.float32),
         jax.ShapeDtypeStruct((2, N_NODES, 1), jnp.float32)],
    )


def _stage3(y3a, y3b, deg, bd1, wd2):
    def body(ya_ref, yb_ref, deg_ref, b_ref, w_ref, out_ref):
        di = _dinv(deg_ref)
        ycat = jnp.concatenate(
            [ya_ref[0], yb_ref[0], ya_ref[1], yb_ref[1]], axis=-1
        )
        h1 = jnp.maximum(ycat * di + b_ref[...], 0.0)
        u4 = jnp.dot(h1, w_ref[...], preferred_element_type=jnp.float32) * di
        out_ref[0] = u4[:, :64]
        out_ref[1] = u4[:, 64:]

    return _tc_call(
        body, (y3a, y3b, deg, bd1, wd2),
        [_splitspec(64), _splitspec(64), _degspec(), _fullspec(1, 256),
         _fullspec(256, 128)],
        _splitspec(64),
        jax.ShapeDtypeStruct((2, N_NODES, 64), jnp.float32),
    )


def _stage4(y4, deg, bd2, wdf, bdf):
    def body(y_ref, deg_ref, b_ref, w_ref, bo_ref, out_ref):
        di = _dinv(deg_ref)
        h2 = _cat(y_ref) * di + b_ref[...]
        out_ref[...] = jnp.dot(h2, w_ref[...], preferred_element_type=jnp.float32) + bo_ref[...]

    return _tc_call(
        body, (y4, deg, bd2, wdf, bdf),
        [_splitspec(64), _degspec(), _fullspec(1, 128), _fullspec(128, 1024),
         _fullspec(1, 1024)],
        _rowspec(1024),
        jax.ShapeDtypeStruct((N_NODES, 1024), jnp.float32),
    )


def kernel(x, edge_index, W1, b1, W2, b2, Wf, bf, Wd1, bd1, Wd2, bd2, Wdf, bdf, Wl, bl):
    n = N_NODES
    src = edge_index[0].astype(jnp.int32)
    dst = edge_index[1].astype(jnp.int32)

    npad = E_PAD - N_EDGES
    pad_src = (jnp.arange(npad, dtype=jnp.int32)) % n
    pad_dst = n + (jnp.arange(npad, dtype=jnp.int32)) % PAD_ROWS
    srcp = jnp.concatenate([src, pad_src])
    dstp = jnp.concatenate([dst, pad_dst])
    srcadj = jnp.stack([srcp, srcp + n]).reshape(2, NS, NCHUNK, K)
    dstp3 = dstp.reshape(NS, NCHUNK, K)
    srcE = srcp.reshape(NC * NS, EPW)
    dstE = dstp.reshape(NC * NS, EPW)

    b1r = b1.reshape(1, -1)
    b2r = b2.reshape(1, -1)
    bfr = bf.reshape(1, -1)
    bd1r = bd1.reshape(1, -1)
    bd2r = bd2.reshape(1, -1)
    bdfr = bdf.reshape(1, -1)
    blr = bl.reshape(1, 1)

    onesz = jnp.concatenate(
        [jnp.ones((n, 16), jnp.float32), jnp.zeros((n, 16), jnp.float32)]
    )
    deg = _make_degree()(onesz, dstp3)

    u1 = _stage0(x, W1, deg)
    y1 = _make_propagate(64, 2)(u1.reshape(2 * n, 64), srcadj, dstp3)
    u2 = _stage1(y1, deg, b1r, W2)
    y2 = _make_propagate(32, 1)(u2.reshape(2 * n, 32), srcadj, dstp3)
    u3, pq = _stage2(y2, deg, b2r, Wf, bfr, Wd1, Wl, blr)
    y3a = _make_propagate(64, 2)(u3[0].reshape(2 * n, 64), srcadj, dstp3)
    y3b = _make_propagate(64, 2)(u3[1].reshape(2 * n, 64), srcadj, dstp3)
    u4 = _stage3(y3a, y3b, deg, bd1r, Wd2)
    y4 = _make_propagate(64, 2)(u4.reshape(2 * n, 64), srcadj, dstp3)
    x_hat = _stage4(y4, deg, bd2r, Wdf, bdfr)

    ep = _make_edgeprob()(pq.reshape(2, n), srcE, dstE)
    edge_probs = ep[:N_EDGES].reshape(N_EDGES, 1)
    return (x_hat, edge_probs)


# speedup vs baseline: 21.2099x; 1.1592x over previous
import functools

import jax
import jax.numpy as jnp
from jax import lax
from jax.experimental import pallas as pl
from jax.experimental.pallas import tpu as pltpu
from jax.experimental.pallas import tpu_sc as plsc

N_NODES = 10000
N_EDGES = 320000

NC = 2
NS = 16
K = 128
NBUF = 2

NCHUNK = 160
E_PAD = NS * NCHUNK * K
EPT = E_PAD // NS
EPW = E_PAD // (NC * NS)

PAD_ROWS = 16
NPS = 624
NREM = N_NODES - NS * NPS


def _sc_mesh():
    return plsc.VectorSubcoreMesh(core_axis_name="c", subcore_axis_name="s")


def _sc_params():
    return pltpu.CompilerParams(use_tc_tiling_on_sc=False, needs_layout_passes=False)


@functools.lru_cache(maxsize=None)
def _make_propagate(ch, kc, phases, nbuf):
    nchunk = EPT // kc
    nch_p = nchunk // phases
    assert nch_p % nbuf == 0

    @functools.partial(
        pl.kernel,
        out_type=jax.ShapeDtypeStruct((NC, N_NODES, ch), jnp.float32),
        mesh=_sc_mesh(),
        compiler_params=_sc_params(),
        scratch_types=[
            pltpu.VMEM_SHARED((N_NODES + PAD_ROWS, ch), jnp.float32),
            pltpu.VMEM((nch_p, kc), jnp.int32),
            pltpu.VMEM((nch_p, kc), jnp.int32),
        ]
        + [pltpu.VMEM((kc, ch), jnp.float32) for _ in range(nbuf)]
        + [pltpu.SemaphoreType.DMA for _ in range(nbuf)],
    )
    def prop(u_hbm, srcadj_hbm, dst_hbm, out_hbm, acc, isall, idall, *rest):
        bufs = rest[:nbuf]
        sems = rest[nbuf:]
        c = lax.axis_index("c")
        s = lax.axis_index("s")
        pltpu.sync_copy(
            u_hbm.at[pl.ds(c * N_NODES + s * NPS, NPS)],
            acc.at[pl.ds(s * NPS, NPS)],
        )

        @pl.when(s == NS - 1)
        def _():
            pltpu.sync_copy(
                u_hbm.at[pl.ds(c * N_NODES + NS * NPS, NREM)],
                acc.at[pl.ds(NS * NPS, NREM)],
            )

        for h in range(phases):
            pltpu.sync_copy(srcadj_hbm.at[c, s, pl.ds(h * nch_p, nch_p)], isall)
            pltpu.sync_copy(dst_hbm.at[s, pl.ds(h * nch_p, nch_p)], idall)

            for b in range(nbuf):
                pltpu.async_copy(u_hbm.at[isall.at[b]], bufs[b], sems[b])

            if h == 0:
                plsc.subcore_barrier()

            def outer(g, carry):
                for b in range(nbuf):
                    i = g * nbuf + b
                    pltpu.make_async_copy(
                        u_hbm.at[isall.at[i]], bufs[b], sems[b]
                    ).wait()
                    pltpu.async_copy(
                        bufs[b], acc.at[idall.at[i]], sems[b], add=True
                    ).wait()

                    @pl.when(i + nbuf < nch_p)
                    def _():
                        pltpu.async_copy(
                            u_hbm.at[isall.at[i + nbuf]], bufs[b], sems[b]
                        )

                return carry

            lax.fori_loop(0, nch_p // nbuf, outer, 0)

        plsc.subcore_barrier()
        pltpu.sync_copy(
            acc.at[pl.ds(s * NPS, NPS)],
            out_hbm.at[c, pl.ds(s * NPS, NPS)],
        )

        @pl.when(s == NS - 1)
        def _():
            pltpu.sync_copy(
                acc.at[pl.ds(NS * NPS, NREM)],
                out_hbm.at[c, pl.ds(NS * NPS, NREM)],
            )

    return prop


def _make_degree():
    halfc = NCHUNK // NC

    @functools.partial(
        pl.kernel,
        out_type=jax.ShapeDtypeStruct((NC, N_NODES, 16), jnp.float32),
        mesh=_sc_mesh(),
        compiler_params=_sc_params(),
        scratch_types=[
            pltpu.VMEM_SHARED((N_NODES + PAD_ROWS, 16), jnp.float32),
            pltpu.VMEM((halfc, K), jnp.int32),
            pltpu.VMEM((K, 16), jnp.float32),
        ]
        + [pltpu.SemaphoreType.DMA for _ in range(NBUF)],
    )
    def degk(onesz_hbm, dst_hbm, out_hbm, acc, idall, ones, *sems):
        c = lax.axis_index("c")
        s = lax.axis_index("s")
        pltpu.sync_copy(
            onesz_hbm.at[pl.ds(c * N_NODES + s * NPS, NPS)],
            acc.at[pl.ds(s * NPS, NPS)],
        )

        @pl.when(s == NS - 1)
        def _():
            pltpu.sync_copy(
                onesz_hbm.at[pl.ds(c * N_NODES + NS * NPS, NREM)],
                acc.at[pl.ds(NS * NPS, NREM)],
            )

        pltpu.sync_copy(onesz_hbm.at[pl.ds(0, K)], ones)
        pltpu.sync_copy(dst_hbm.at[s, pl.ds(c * halfc, halfc)], idall)
        plsc.subcore_barrier()

        def outer(g, carry):
            for b in range(NBUF):
                i = g * NBUF + b

                @pl.when(g > 0)
                def _():
                    pltpu.make_async_copy(ones, acc.at[idall.at[i]], sems[b]).wait()

                pltpu.async_copy(ones, acc.at[idall.at[i]], sems[b], add=True)
            return carry

        lax.fori_loop(0, halfc // NBUF, outer, 0)
        for b in range(NBUF):
            pltpu.make_async_copy(ones, acc.at[idall.at[b]], sems[b]).wait()

        plsc.subcore_barrier()
        pltpu.sync_copy(
            acc.at[pl.ds(s * NPS, NPS)],
            out_hbm.at[c, pl.ds(s * NPS, NPS)],
        )

        @pl.when(s == NS - 1)
        def _():
            pltpu.sync_copy(
                acc.at[pl.ds(NS * NPS, NREM)],
                out_hbm.at[c, pl.ds(NS * NPS, NREM)],
            )

    return degk


def _make_edgeprob():
    @functools.partial(
        pl.kernel,
        out_type=jax.ShapeDtypeStruct((E_PAD,), jnp.float32),
        mesh=_sc_mesh(),
        compiler_params=_sc_params(),
        scratch_types=[
            pltpu.VMEM((N_NODES,), jnp.float32),
            pltpu.VMEM((N_NODES + PAD_ROWS,), jnp.float32),
            pltpu.VMEM((EPW,), jnp.int32),
            pltpu.VMEM((EPW,), jnp.int32),
            pltpu.VMEM((EPW,), jnp.float32),
        ],
    )
    def eprob(pq_hbm, src_hbm, dst_hbm, out_hbm, pv, qv, is_, id_, ob):
        c = lax.axis_index("c")
        s = lax.axis_index("s")
        w = s * NC + c
        pltpu.sync_copy(pq_hbm.at[0], pv)
        pltpu.sync_copy(pq_hbm.at[1], qv.at[pl.ds(0, N_NODES)])
        pltpu.sync_copy(src_hbm.at[w], is_)
        pltpu.sync_copy(dst_hbm.at[w], id_)

        def sub(j, carry):
            sv = is_[pl.ds(j * 16, 16)]
            dv = id_[pl.ds(j * 16, 16)]
            a = plsc.load_gather(pv, [sv])
            b = plsc.load_gather(qv, [dv])
            t = a + b
            ob[pl.ds(j * 16, 16)] = 1.0 / (1.0 + jnp.exp(-t))
            return carry

        lax.fori_loop(0, EPW // 16, sub, 0)
        pltpu.sync_copy(ob, out_hbm.at[pl.ds(w * EPW, EPW)])

    return eprob


BN = 1000


def _dinv(deg_ref):
    d = deg_ref[0][:, 0:1] + deg_ref[1][:, 0:1]
    return lax.rsqrt(d)


def _cat(y_ref):
    return jnp.concatenate([y_ref[0], y_ref[1]], axis=-1)


def _tc_call(body, in_arrays, in_specs, out_specs, out_shape):
    return pl.pallas_call(
        body,
        grid=(N_NODES // BN,),
        in_specs=in_specs,
        out_specs=out_specs,
        out_shape=out_shape,
    )(*in_arrays)


def _rowspec(c):
    return pl.BlockSpec((BN, c), lambda i: (i, 0))


def _fullspec(r, c):
    return pl.BlockSpec((r, c), lambda i: (0, 0))


def _splitspec(ch):
    return pl.BlockSpec((2, BN, ch), lambda i: (0, i, 0))


def _degspec():
    return pl.BlockSpec((2, BN, 16), lambda i: (0, i, 0))


def _stage0(x, w1, deg):
    def body(x_ref, w_ref, deg_ref, out_ref):
        u = jnp.dot(x_ref[...], w_ref[...], preferred_element_type=jnp.float32)
        u = u * _dinv(deg_ref)
        out_ref[0] = u[:, :64]
        out_ref[1] = u[:, 64:]

    return _tc_call(
        body, (x, w1, deg),
        [_rowspec(128), _fullspec(128, 128), _degspec()],
        _splitspec(64),
        jax.ShapeDtypeStruct((2, N_NODES, 64), jnp.float32),
    )


def _stage1(y1, deg, b1, w2):
    def body(y_ref, deg_ref, b_ref, w_ref, out_ref):
        di = _dinv(deg_ref)
        z1 = jnp.maximum(_cat(y_ref) * di + b_ref[...], 0.0)
        u2 = jnp.dot(z1, w_ref[...], preferred_element_type=jnp.float32) * di
        out_ref[0] = u2[:, :32]
        out_ref[1] = u2[:, 32:]

    return _tc_call(
        body, (y1, deg, b1, w2),
        [_splitspec(64), _degspec(), _fullspec(1, 128), _fullspec(128, 64)],
        _splitspec(32),
        jax.ShapeDtypeStruct((2, N_NODES, 32), jnp.float32),
    )


def _stage2(y2, deg, b2, wf, bf, wd1, wl, bl):
    def body(y_ref, deg_ref, b2_ref, wf_ref, bf_ref, wd1_ref, wl_ref, bl_ref,
             u3_ref, pq_ref):
        di = _dinv(deg_ref)
        z = _cat(y_ref) * di + b2_ref[...]
        z = jnp.dot(z, wf_ref[...], preferred_element_type=jnp.float32) + bf_ref[...]
        u3 = jnp.dot(z, wd1_ref[...], preferred_element_type=jnp.float32) * di
        u3_ref[0, 0] = u3[:, 0:64]
        u3_ref[1, 0] = u3[:, 64:128]
        u3_ref[0, 1] = u3[:, 128:192]
        u3_ref[1, 1] = u3[:, 192:256]
        pq_ref[0] = jnp.dot(z, wl_ref[...][:64, :], preferred_element_type=jnp.float32)
        pq_ref[1] = jnp.dot(z, wl_ref[...][64:, :], preferred_element_type=jnp.float32) + bl_ref[...]

    return _tc_call(
        body, (y2, deg, b2, wf, bf, wd1, wl, bl),
        [_splitspec(32), _degspec(), _fullspec(1, 64), _fullspec(64, 64),
         _fullspec(1, 64), _fullspec(64, 256), _fullspec(128, 1), _fullspec(1, 1)],
        [pl.BlockSpec((2, 2, BN, 64), lambda i: (0, 0, i, 0)),
         pl.BlockSpec((2, BN, 1), lambda i: (0, i, 0))],
        [jax.ShapeDtypeStruct((2, 2, N_NODES, 64), jnp.float32),
         jax.ShapeDtypeStruct((2, N_NODES, 1), jnp.float32)],
    )


def _stage3(y3a, y3b, deg, bd1, wd2):
    def body(ya_ref, yb_ref, deg_ref, b_ref, w_ref, out_ref):
        di = _dinv(deg_ref)
        ycat = jnp.concatenate(
            [ya_ref[0], yb_ref[0], ya_ref[1], yb_ref[1]], axis=-1
        )
        h1 = jnp.maximum(ycat * di + b_ref[...], 0.0)
        u4 = jnp.dot(h1, w_ref[...], preferred_element_type=jnp.float32) * di
        out_ref[0] = u4[:, :64]
        out_ref[1] = u4[:, 64:]

    return _tc_call(
        body, (y3a, y3b, deg, bd1, wd2),
        [_splitspec(64), _splitspec(64), _degspec(), _fullspec(1, 256),
         _fullspec(256, 128)],
        _splitspec(64),
        jax.ShapeDtypeStruct((2, N_NODES, 64), jnp.float32),
    )


def _stage4(y4, deg, bd2, wdf, bdf):
    def body(y_ref, deg_ref, b_ref, w_ref, bo_ref, out_ref):
        di = _dinv(deg_ref)
        h2 = _cat(y_ref) * di + b_ref[...]
        out_ref[...] = jnp.dot(h2, w_ref[...], preferred_element_type=jnp.float32) + bo_ref[...]

    return _tc_call(
        body, (y4, deg, bd2, wdf, bdf),
        [_splitspec(64), _degspec(), _fullspec(1, 128), _fullspec(128, 1024),
         _fullspec(1, 1024)],
        _rowspec(1024),
        jax.ShapeDtypeStruct((N_NODES, 1024), jnp.float32),
    )


def kernel(x, edge_index, W1, b1, W2, b2, Wf, bf, Wd1, bd1, Wd2, bd2, Wdf, bdf, Wl, bl):
    n = N_NODES
    src = edge_index[0].astype(jnp.int32)
    dst = edge_index[1].astype(jnp.int32)

    npad = E_PAD - N_EDGES
    pad_src = (jnp.arange(npad, dtype=jnp.int32)) % n
    pad_dst = n + (jnp.arange(npad, dtype=jnp.int32)) % PAD_ROWS
    srcp = jnp.concatenate([src, pad_src])
    dstp = jnp.concatenate([dst, pad_dst])
    srcadj2 = jnp.stack([srcp, srcp + n])
    srcadj = srcadj2.reshape(2, NS, NCHUNK, K)
    dstp3 = dstp.reshape(NS, NCHUNK, K)
    srcadj64 = srcadj2.reshape(2, NS, EPT // 64, 64)
    dst64 = dstp.reshape(NS, EPT // 64, 64)
    srcE = srcp.reshape(NC * NS, EPW)
    dstE = dstp.reshape(NC * NS, EPW)

    b1r = b1.reshape(1, -1)
    b2r = b2.reshape(1, -1)
    bfr = bf.reshape(1, -1)
    bd1r = bd1.reshape(1, -1)
    bd2r = bd2.reshape(1, -1)
    bdfr = bdf.reshape(1, -1)
    blr = bl.reshape(1, 1)

    onesz = jnp.concatenate(
        [jnp.ones((n, 16), jnp.float32), jnp.zeros((n, 16), jnp.float32)]
    )
    deg = _make_degree()(onesz, dstp3)

    u1 = _stage0(x, W1, deg)
    y1 = _make_propagate(64, 64, 4, 5)(u1.reshape(2 * n, 64), srcadj64, dst64)
    u2 = _stage1(y1, deg, b1r, W2)
    y2 = _make_propagate(32, 64, 4, 8)(u2.reshape(2 * n, 32), srcadj64, dst64)
    u3, pq = _stage2(y2, deg, b2r, Wf, bfr, Wd1, Wl, blr)
    y3a = _make_propagate(64, 64, 4, 5)(u3[0].reshape(2 * n, 64), srcadj64, dst64)
    y3b = _make_propagate(64, 64, 4, 5)(u3[1].reshape(2 * n, 64), srcadj64, dst64)
    u4 = _stage3(y3a, y3b, deg, bd1r, Wd2)
    y4 = _make_propagate(64, 64, 4, 5)(u4.reshape(2 * n, 64), srcadj64, dst64)
    x_hat = _stage4(y4, deg, bd2r, Wdf, bdfr)

    ep = _make_edgeprob()(pq.reshape(2, n), srcE, dstE)
    edge_probs = ep[:N_EDGES].reshape(N_EDGES, 1)
    return (x_hat, edge_probs)
